# baseline (speedup 1.0000x reference)
import functools

import jax
import jax.numpy as jnp
from jax import lax
from jax.experimental import pallas as pl
from jax.experimental.pallas import tpu as pltpu
from jax.experimental.pallas import tpu_sc as plsc

_NC = 2
_NS = 16
_NW = _NC * _NS
_L = 16

_TOTAL = 2048 * 2048
_PER_W = _TOTAL // _NW
_CHUNK = 4096
_NCH = _PER_W // _CHUNK

_NB = 64
_PW = 64 + 2 * _NB

_LN2 = 0.6931471805599453


def _vlog(x):
    b = lax.bitcast_convert_type(x, jnp.int32)
    e = (b >> 23) - 127
    mb = (b & 0x7FFFFF) | (127 << 23)
    m = lax.bitcast_convert_type(mb, jnp.float32)
    big = m > 1.4142135
    m = jnp.where(big, m * 0.5, m)
    e = jnp.where(big, e + 1, e)
    ef = e.astype(jnp.float32)
    s = (m - 1.0) / (m + 1.0)
    z = s * s
    poly = 1.0 + z * (1.0 / 3.0 + z * (1.0 / 5.0 + z * (1.0 / 7.0)))
    return ef * _LN2 + (2.0 * s) * poly


_mesh = plsc.VectorSubcoreMesh(core_axis_name="c", subcore_axis_name="s")


@functools.partial(
    pl.kernel,
    out_type=jax.ShapeDtypeStruct((_NW * _PW,), jnp.float32),
    mesh=_mesh,
    compiler_params=pltpu.CompilerParams(needs_layout_passes=False),
    scratch_types=[
        pltpu.VMEM((2, _CHUNK), jnp.float32),
        pltpu.VMEM((2, _CHUNK), jnp.float32),
        pltpu.VMEM((2, _CHUNK), jnp.float32),
        pltpu.VMEM((_NB * _L,), jnp.float32),
        pltpu.VMEM((_NB * _L,), jnp.float32),
        pltpu.VMEM((_NB * _L,), jnp.float32),
        pltpu.VMEM((_NB * _L,), jnp.float32),
        pltpu.VMEM((_PW,), jnp.float32),
        pltpu.SemaphoreType.DMA,
        pltpu.SemaphoreType.DMA,
    ],
)
def _pass1(pred_hbm, gt_hbm, mask_hbm, out_hbm, pbuf, gbuf, mbuf, hcnt, hsum,
           hcnt1, hsum1, obuf, sem0, sem1):
    wid = lax.axis_index("s") * _NC + lax.axis_index("c")
    lane = lax.iota(jnp.int32, _L)
    zero = jnp.zeros((_L,), jnp.float32)
    sems = (sem0, sem1)

    def zinit(b, _):
        hcnt[pl.ds(b * _L, _L)] = zero
        hsum[pl.ds(b * _L, _L)] = zero
        hcnt1[pl.ds(b * _L, _L)] = zero
        hsum1[pl.ds(b * _L, _L)] = zero
        return 0

    lax.fori_loop(0, _NB, zinit, 0)

    def start(c, slot):
        base = wid * _PER_W + c * _CHUNK
        sl = pl.ds(base, _CHUNK)
        pltpu.async_copy(pred_hbm.at[sl], pbuf.at[slot], sems[slot])
        pltpu.async_copy(gt_hbm.at[sl], gbuf.at[slot], sems[slot])
        pltpu.async_copy(mask_hbm.at[sl], mbuf.at[slot], sems[slot])

    def drain(slot):
        sl = pl.ds(0, _CHUNK)
        pltpu.make_async_copy(pred_hbm.at[sl], pbuf.at[slot], sems[slot]).wait()
        pltpu.make_async_copy(gt_hbm.at[sl], gbuf.at[slot], sems[slot]).wait()
        pltpu.make_async_copy(mask_hbm.at[sl], mbuf.at[slot], sems[slot]).wait()

    def compute(slot, carry):
        def elem_body(j, acc):
            pc, nc, ps, ns = acc
            for u in range(2):
                off = j * (2 * _L) + u * _L
                p = pbuf[slot, pl.ds(off, _L)]
                g = gbuf[slot, pl.ds(off, _L)]
                m = mbuf[slot, pl.ds(off, _L)]
                pos = g * m
                neg = m - pos
                y = jnp.where(g > 0.5, p, 1.0 - p)
                loss = -_vlog(y + 1e-37) * jnp.exp(-y)
                nl = loss * neg
                bin_i = jnp.minimum((p * float(_NB)).astype(jnp.int32), _NB - 1)
                idx = (_NB - 1 - bin_i) * _L + lane
                isneg = neg > 0.5
                hc, hs = ((hcnt, hsum), (hcnt1, hsum1))[u]
                plsc.addupdate_scatter(hc, [idx], neg, mask=isneg)
                plsc.addupdate_scatter(hs, [idx], nl, mask=isneg)
                pc, nc, ps, ns = pc + pos, nc + neg, ps + loss * pos, ns + nl
            return (pc, nc, ps, ns)

        return lax.fori_loop(0, _CHUNK // (2 * _L), elem_body, carry)

    start(0, 0)

    def chunk_pair(c2, carry):
        c = 2 * c2
        start(c + 1, 1)
        drain(0)
        carry = compute(0, carry)

        @pl.when(c2 < _NCH // 2 - 1)
        def _():
            start(c + 2, 0)

        drain(1)
        return compute(1, carry)

    pc, nc, ps, ns = lax.fori_loop(
        0, _NCH // 2, chunk_pair, (zero, zero, zero, zero))

    obuf[pl.ds(0, _L)] = pc
    obuf[pl.ds(16, _L)] = nc
    obuf[pl.ds(32, _L)] = ps
    obuf[pl.ds(48, _L)] = ns

    stride = lane * _L
    for grp in range(_NB // _L):
        cacc = zero
        sacc = zero
        for l in range(_L):
            gi = stride + (grp * _L * _L + l)
            cacc = cacc + plsc.load_gather(hcnt, [gi]) + plsc.load_gather(
                hcnt1, [gi])
            sacc = sacc + plsc.load_gather(hsum, [gi]) + plsc.load_gather(
                hsum1, [gi])
        obuf[pl.ds(64 + grp * _L, _L)] = cacc
        obuf[pl.ds(64 + _NB + grp * _L, _L)] = sacc

    pltpu.sync_copy(obuf, out_hbm.at[pl.ds(wid * _PW, _PW)])


@functools.partial(
    pl.kernel,
    out_type=jax.ShapeDtypeStruct((_L,), jnp.float32),
    mesh=_mesh,
    compiler_params=pltpu.CompilerParams(needs_layout_passes=False),
    scratch_types=[
        pltpu.VMEM((_NW * _PW,), jnp.float32),
        pltpu.VMEM((_L,), jnp.float32),
    ],
)
def _pass2(part_hbm, out_hbm, pbuf, obuf):
    wid = lax.axis_index("s") * _NC + lax.axis_index("c")

    @pl.when(wid == 0)
    def _():
        pltpu.sync_copy(part_hbm, pbuf)
        nvec = _PW // _L
        zero = jnp.zeros((_L,), jnp.float32)

        def wbody(w, accs):
            base = w * _PW
            return tuple(
                accs[i] + pbuf[pl.ds(base + i * _L, _L)] for i in range(nvec))

        accs = lax.fori_loop(0, _NW, wbody, tuple(zero for _ in range(nvec)))

        pc = jnp.sum(accs[0])
        nc = jnp.sum(accs[1])
        ps = jnp.sum(accs[2])
        k = jnp.minimum(nc, 3.0 * pc)

        top = jnp.float32(0.0)
        cum = jnp.float32(0.0)
        for grp in range(_NB // _L):
            c16 = accs[4 + grp]
            s16 = accs[4 + _NB // _L + grp]
            excl = plsc.cumsum(c16) - c16
            take = jnp.clip(k - (cum + excl), 0.0, c16)
            top = top + jnp.sum(take * (s16 / jnp.maximum(c16, 1.0)))
            cum = cum + jnp.sum(c16)

        num = jnp.full((_L,), ps + top, jnp.float32)
        den = jnp.full((_L,), pc + k + 1e-6, jnp.float32)
        obuf[...] = num / den
        pltpu.sync_copy(obuf, out_hbm)


def kernel(pred, gt, mask):
    p = pred.reshape(-1)
    g = gt.reshape(-1)
    m = mask.reshape(-1)
    part = _pass1(p, g, m)
    out = _pass2(part)
    return out[0]


# speedup vs baseline: 2.1944x; 2.1944x over previous
import functools

import jax
import jax.numpy as jnp
from jax import lax
from jax.experimental import pallas as pl
from jax.experimental.pallas import tpu as pltpu
from jax.experimental.pallas import tpu_sc as plsc

_NC = 2
_NS = 16
_NW = _NC * _NS
_L = 16

_TOTAL = 2048 * 2048
_PER_W = _TOTAL // _NW
_CHUNK = 4096
_NCH = _PER_W // _CHUNK

_NB = 64
_PW = 64 + 2 * _NB
_SW = 64

_LN2 = 0.6931471805599453


def _vlog(x):
    b = lax.bitcast_convert_type(x, jnp.int32)
    e = (b >> 23) - 127
    mb = (b & 0x7FFFFF) | (127 << 23)
    m = lax.bitcast_convert_type(mb, jnp.float32)
    big = m > 1.4142135
    m = jnp.where(big, m * 0.5, m)
    e = jnp.where(big, e + 1, e)
    ef = e.astype(jnp.float32)
    s = (m - 1.0) / (m + 1.0)
    z = s * s
    poly = 1.0 + z * (1.0 / 3.0 + z * (1.0 / 5.0 + z * (1.0 / 7.0)))
    return ef * _LN2 + (2.0 * s) * poly


_mesh = plsc.VectorSubcoreMesh(core_axis_name="c", subcore_axis_name="s")


@functools.partial(
    pl.kernel,
    out_type=jax.ShapeDtypeStruct((_NW * _SW,), jnp.float32),
    mesh=_mesh,
    compiler_params=pltpu.CompilerParams(needs_layout_passes=False),
    scratch_types=[
        pltpu.VMEM((2, _CHUNK), jnp.float32),
        pltpu.VMEM((2, _CHUNK), jnp.float32),
        pltpu.VMEM((2, _CHUNK), jnp.float32),
        pltpu.VMEM((_SW,), jnp.float32),
        pltpu.SemaphoreType.DMA,
        pltpu.SemaphoreType.DMA,
    ],
)
def _sums(pred_hbm, gt_hbm, mask_hbm, out_hbm, pbuf, gbuf, mbuf, obuf,
          sem0, sem1):
    wid = lax.axis_index("s") * _NC + lax.axis_index("c")
    zero = jnp.zeros((_L,), jnp.float32)
    sems = (sem0, sem1)

    def start(c, slot):
        base = wid * _PER_W + c * _CHUNK
        sl = pl.ds(base, _CHUNK)
        pltpu.async_copy(pred_hbm.at[sl], pbuf.at[slot], sems[slot])
        pltpu.async_copy(gt_hbm.at[sl], gbuf.at[slot], sems[slot])
        pltpu.async_copy(mask_hbm.at[sl], mbuf.at[slot], sems[slot])

    def drain(slot):
        sl = pl.ds(0, _CHUNK)
        pltpu.make_async_copy(pred_hbm.at[sl], pbuf.at[slot], sems[slot]).wait()
        pltpu.make_async_copy(gt_hbm.at[sl], gbuf.at[slot], sems[slot]).wait()
        pltpu.make_async_copy(mask_hbm.at[sl], mbuf.at[slot], sems[slot]).wait()

    def compute(slot, carry):
        def elem_body(j, acc):
            pc, nc, ps, ns = acc
            for u in range(2):
                off = j * (2 * _L) + u * _L
                p = pbuf[slot, pl.ds(off, _L)]
                g = gbuf[slot, pl.ds(off, _L)]
                m = mbuf[slot, pl.ds(off, _L)]
                pos = g * m
                neg = m - pos
                y = jnp.where(g > 0.5, p, 1.0 - p)
                loss = -_vlog(y + 1e-37) * jnp.exp(-y)
                pc, nc, ps, ns = (pc + pos, nc + neg, ps + loss * pos,
                                  ns + loss * neg)
            return (pc, nc, ps, ns)

        return lax.fori_loop(0, _CHUNK // (2 * _L), elem_body, carry)

    start(0, 0)

    def chunk_pair(c2, carry):
        c = 2 * c2
        start(c + 1, 1)
        drain(0)
        carry = compute(0, carry)

        @pl.when(c2 < _NCH // 2 - 1)
        def _():
            start(c + 2, 0)

        drain(1)
        return compute(1, carry)

    pc, nc, ps, ns = lax.fori_loop(
        0, _NCH // 2, chunk_pair, (zero, zero, zero, zero))

    obuf[pl.ds(0, _L)] = pc
    obuf[pl.ds(16, _L)] = nc
    obuf[pl.ds(32, _L)] = ps
    obuf[pl.ds(48, _L)] = ns
    pltpu.sync_copy(obuf, out_hbm.at[pl.ds(wid * _SW, _SW)])


@functools.partial(
    pl.kernel,
    out_type=jax.ShapeDtypeStruct((_NW * _PW,), jnp.float32),
    mesh=_mesh,
    compiler_params=pltpu.CompilerParams(needs_layout_passes=False),
    scratch_types=[
        pltpu.VMEM((2, _CHUNK), jnp.float32),
        pltpu.VMEM((2, _CHUNK), jnp.float32),
        pltpu.VMEM((2, _CHUNK), jnp.float32),
        pltpu.VMEM((_NB * _L,), jnp.float32),
        pltpu.VMEM((_NB * _L,), jnp.float32),
        pltpu.VMEM((_PW,), jnp.float32),
        pltpu.VMEM((_NW * _SW,), jnp.float32),
        pltpu.SemaphoreType.DMA,
        pltpu.SemaphoreType.DMA,
    ],
)
def _hist(pred_hbm, gt_hbm, mask_hbm, part1_hbm, out_hbm, pbuf, gbuf, mbuf,
          hcnt, hsum, obuf, p1buf, sem0, sem1):
    wid = lax.axis_index("s") * _NC + lax.axis_index("c")
    lane = lax.iota(jnp.int32, _L)
    zero = jnp.zeros((_L,), jnp.float32)
    sems = (sem0, sem1)

    pltpu.sync_copy(part1_hbm, p1buf)

    def redw(w, acc):
        pcv, ncv = acc
        return (pcv + p1buf[pl.ds(w * _SW, _L)],
                ncv + p1buf[pl.ds(w * _SW + 16, _L)])

    pcv, ncv = lax.fori_loop(0, _NW, redw, (zero, zero))
    heavy = jnp.sum(ncv) > 3.0 * jnp.sum(pcv)

    @pl.when(heavy)
    def _():
        def zinit(b, _):
            hcnt[pl.ds(b * _L, _L)] = zero
            hsum[pl.ds(b * _L, _L)] = zero
            return 0

        lax.fori_loop(0, _NB, zinit, 0)

        def start(c, slot):
            base = wid * _PER_W + c * _CHUNK
            sl = pl.ds(base, _CHUNK)
            pltpu.async_copy(pred_hbm.at[sl], pbuf.at[slot], sems[slot])
            pltpu.async_copy(gt_hbm.at[sl], gbuf.at[slot], sems[slot])
            pltpu.async_copy(mask_hbm.at[sl], mbuf.at[slot], sems[slot])

        def drain(slot):
            sl = pl.ds(0, _CHUNK)
            pltpu.make_async_copy(
                pred_hbm.at[sl], pbuf.at[slot], sems[slot]).wait()
            pltpu.make_async_copy(
                gt_hbm.at[sl], gbuf.at[slot], sems[slot]).wait()
            pltpu.make_async_copy(
                mask_hbm.at[sl], mbuf.at[slot], sems[slot]).wait()

        def compute(slot, carry):
            def elem_body(j, acc):
                pc, nc, ps, ns = acc
                for u in range(2):
                    off = j * (2 * _L) + u * _L
                    p = pbuf[slot, pl.ds(off, _L)]
                    g = gbuf[slot, pl.ds(off, _L)]
                    m = mbuf[slot, pl.ds(off, _L)]
                    pos = g * m
                    neg = m - pos
                    y = jnp.where(g > 0.5, p, 1.0 - p)
                    loss = -_vlog(y + 1e-37) * jnp.exp(-y)
                    nl = loss * neg
                    bin_i = jnp.minimum(
                        (p * float(_NB)).astype(jnp.int32), _NB - 1)
                    idx = (_NB - 1 - bin_i) * _L + lane
                    plsc.addupdate_scatter(hcnt, [idx], neg)
                    plsc.addupdate_scatter(hsum, [idx], nl)
                    pc, nc, ps, ns = pc + pos, nc + neg, ps + loss * pos, ns + nl
                return (pc, nc, ps, ns)

            return lax.fori_loop(0, _CHUNK // (2 * _L), elem_body, carry)

        start(0, 0)

        def chunk_pair(c2, carry):
            c = 2 * c2
            start(c + 1, 1)
            drain(0)
            carry = compute(0, carry)

            @pl.when(c2 < _NCH // 2 - 1)
            def _():
                start(c + 2, 0)

            drain(1)
            return compute(1, carry)

        pc, nc, ps, ns = lax.fori_loop(
            0, _NCH // 2, chunk_pair, (zero, zero, zero, zero))

        obuf[pl.ds(0, _L)] = pc
        obuf[pl.ds(16, _L)] = nc
        obuf[pl.ds(32, _L)] = ps
        obuf[pl.ds(48, _L)] = ns

        stride = lane * _L
        for grp in range(_NB // _L):
            cacc = zero
            sacc = zero
            for l in range(_L):
                gi = stride + (grp * _L * _L + l)
                cacc = cacc + plsc.load_gather(hcnt, [gi])
                sacc = sacc + plsc.load_gather(hsum, [gi])
            obuf[pl.ds(64 + grp * _L, _L)] = cacc
            obuf[pl.ds(64 + _NB + grp * _L, _L)] = sacc

        pltpu.sync_copy(obuf, out_hbm.at[pl.ds(wid * _PW, _PW)])


@functools.partial(
    pl.kernel,
    out_type=jax.ShapeDtypeStruct((_L,), jnp.float32),
    mesh=_mesh,
    compiler_params=pltpu.CompilerParams(needs_layout_passes=False),
    scratch_types=[
        pltpu.VMEM((_NW * _SW,), jnp.float32),
        pltpu.VMEM((_NW * _PW,), jnp.float32),
        pltpu.VMEM((_L,), jnp.float32),
    ],
)
def _final(part1_hbm, part2_hbm, out_hbm, p1buf, p2buf, obuf):
    wid = lax.axis_index("s") * _NC + lax.axis_index("c")

    @pl.when(wid == 0)
    def _():
        zero = jnp.zeros((_L,), jnp.float32)
        pltpu.sync_copy(part1_hbm, p1buf)

        def redw(w, acc):
            base = w * _SW
            return tuple(
                acc[i] + p1buf[pl.ds(base + i * _L, _L)] for i in range(4))

        a1 = lax.fori_loop(0, _NW, redw, (zero, zero, zero, zero))
        pc = jnp.sum(a1[0])
        nc = jnp.sum(a1[1])
        ps = jnp.sum(a1[2])
        ns = jnp.sum(a1[3])
        heavy = nc > 3.0 * pc

        @pl.when(jnp.logical_not(heavy))
        def _():
            num = jnp.full((_L,), ps + ns, jnp.float32)
            den = jnp.full((_L,), pc + nc + 1e-6, jnp.float32)
            obuf[...] = num / den

        @pl.when(heavy)
        def _():
            pltpu.sync_copy(part2_hbm, p2buf)
            nvec = _PW // _L

            def wbody(w, accs):
                base = w * _PW
                return tuple(
                    accs[i] + p2buf[pl.ds(base + i * _L, _L)]
                    for i in range(nvec))

            accs = lax.fori_loop(0, _NW, wbody,
                                 tuple(zero for _ in range(nvec)))
            k = jnp.minimum(nc, 3.0 * pc)

            top = jnp.float32(0.0)
            cum = jnp.float32(0.0)
            for grp in range(_NB // _L):
                c16 = accs[4 + grp]
                s16 = accs[4 + _NB // _L + grp]
                excl = plsc.cumsum(c16) - c16
                take = jnp.clip(k - (cum + excl), 0.0, c16)
                top = top + jnp.sum(take * (s16 / jnp.maximum(c16, 1.0)))
                cum = cum + jnp.sum(c16)

            num = jnp.full((_L,), ps + top, jnp.float32)
            den = jnp.full((_L,), pc + k + 1e-6, jnp.float32)
            obuf[...] = num / den

        pltpu.sync_copy(obuf, out_hbm)


def kernel(pred, gt, mask):
    p = pred.reshape(-1)
    g = gt.reshape(-1)
    m = mask.reshape(-1)
    part1 = _sums(p, g, m)
    part2 = _hist(p, g, m, part1)
    out = _final(part1, part2)
    return out[0]


# speedup vs baseline: 2.9369x; 1.3384x over previous
import functools

import jax
import jax.numpy as jnp
from jax import lax
from jax.experimental import pallas as pl
from jax.experimental.pallas import tpu as pltpu
from jax.experimental.pallas import tpu_sc as plsc

_NC = 2
_NS = 16
_NW = _NC * _NS
_L = 16

_ROWS = 2048
_COLS = 2048
_RW = _ROWS // _NW
_RC = 8
_NCH = _RW // _RC

_NB = 64
_PW = 64 + 2 * _NB
_SW = 64

_LN2 = 0.6931471805599453


def _vlog(x):
    b = lax.bitcast_convert_type(x, jnp.int32)
    e = (b >> 23) - 127
    mb = (b & 0x7FFFFF) | (127 << 23)
    m = lax.bitcast_convert_type(mb, jnp.float32)
    big = m > 1.4142135
    m = jnp.where(big, m * 0.5, m)
    e = jnp.where(big, e + 1, e)
    ef = e.astype(jnp.float32)
    s = (m - 1.0) / (m + 1.0)
    z = s * s
    poly = 1.0 + z * (1.0 / 3.0 + z * (1.0 / 5.0 + z * (1.0 / 7.0)))
    return ef * _LN2 + (2.0 * s) * poly


_mesh = plsc.VectorSubcoreMesh(core_axis_name="c", subcore_axis_name="s")


@functools.partial(
    pl.kernel,
    out_type=jax.ShapeDtypeStruct((_NW * _SW,), jnp.float32),
    mesh=_mesh,
    compiler_params=pltpu.CompilerParams(needs_layout_passes=False),
    scratch_types=[
        pltpu.VMEM((2, _RC, _COLS), jnp.float32),
        pltpu.VMEM((2, _RC, _COLS), jnp.float32),
        pltpu.VMEM((2, _RC, _COLS), jnp.float32),
        pltpu.VMEM((_SW,), jnp.float32),
        pltpu.SemaphoreType.DMA,
        pltpu.SemaphoreType.DMA,
    ],
)
def _sums(pred_hbm, gt_hbm, mask_hbm, out_hbm, pbuf, gbuf, mbuf, obuf,
          sem0, sem1):
    wid = lax.axis_index("s") * _NC + lax.axis_index("c")
    zero = jnp.zeros((_L,), jnp.float32)
    sems = (sem0, sem1)

    def start(c, slot):
        sl = pl.ds(wid * _RW + c * _RC, _RC)
        pltpu.async_copy(pred_hbm.at[sl, :], pbuf.at[slot], sems[slot])
        pltpu.async_copy(gt_hbm.at[sl, :], gbuf.at[slot], sems[slot])
        pltpu.async_copy(mask_hbm.at[sl, :], mbuf.at[slot], sems[slot])

    def drain(slot):
        sl = pl.ds(0, _RC)
        pltpu.make_async_copy(
            pred_hbm.at[sl, :], pbuf.at[slot], sems[slot]).wait()
        pltpu.make_async_copy(
            gt_hbm.at[sl, :], gbuf.at[slot], sems[slot]).wait()
        pltpu.make_async_copy(
            mask_hbm.at[sl, :], mbuf.at[slot], sems[slot]).wait()

    def compute(slot, carry):
        def elem_body(j, acc):
            pc, nc, ps, ns = acc
            for r in range(_RC):
                for u in range(2):
                    off = j * (2 * _L) + u * _L
                    p = pbuf[slot, r, pl.ds(off, _L)]
                    g = gbuf[slot, r, pl.ds(off, _L)]
                    m = mbuf[slot, r, pl.ds(off, _L)]
                    pos = g * m
                    neg = m - pos
                    y = jnp.where(g > 0.5, p, 1.0 - p)
                    loss = -_vlog(y + 1e-37) * jnp.exp(-y)
                    pc, nc, ps, ns = (pc + pos, nc + neg, ps + loss * pos,
                                      ns + loss * neg)
            return (pc, nc, ps, ns)

        return lax.fori_loop(0, _COLS // (2 * _L), elem_body, carry)

    start(0, 0)

    def chunk_pair(c2, carry):
        c = 2 * c2
        start(c + 1, 1)
        drain(0)
        carry = compute(0, carry)

        @pl.when(c2 < _NCH // 2 - 1)
        def _():
            start(c + 2, 0)

        drain(1)
        return compute(1, carry)

    pc, nc, ps, ns = lax.fori_loop(
        0, _NCH // 2, chunk_pair, (zero, zero, zero, zero))

    obuf[pl.ds(0, _L)] = pc
    obuf[pl.ds(16, _L)] = nc
    obuf[pl.ds(32, _L)] = ps
    obuf[pl.ds(48, _L)] = ns
    pltpu.sync_copy(obuf, out_hbm.at[pl.ds(wid * _SW, _SW)])


@functools.partial(
    pl.kernel,
    out_type=jax.ShapeDtypeStruct((_NW * _PW,), jnp.float32),
    mesh=_mesh,
    compiler_params=pltpu.CompilerParams(needs_layout_passes=False),
    scratch_types=[
        pltpu.VMEM((2, _RC, _COLS), jnp.float32),
        pltpu.VMEM((2, _RC, _COLS), jnp.float32),
        pltpu.VMEM((2, _RC, _COLS), jnp.float32),
        pltpu.VMEM((_NB * _L,), jnp.float32),
        pltpu.VMEM((_NB * _L,), jnp.float32),
        pltpu.VMEM((_PW,), jnp.float32),
        pltpu.VMEM((_NW * _SW,), jnp.float32),
        pltpu.SemaphoreType.DMA,
        pltpu.SemaphoreType.DMA,
    ],
)
def _hist(pred_hbm, gt_hbm, mask_hbm, part1_hbm, out_hbm, pbuf, gbuf, mbuf,
          hcnt, hsum, obuf, p1buf, sem0, sem1):
    wid = lax.axis_index("s") * _NC + lax.axis_index("c")
    lane = lax.iota(jnp.int32, _L)
    zero = jnp.zeros((_L,), jnp.float32)
    sems = (sem0, sem1)

    pltpu.sync_copy(part1_hbm, p1buf)

    def redw(w, acc):
        pcv, ncv = acc
        return (pcv + p1buf[pl.ds(w * _SW, _L)],
                ncv + p1buf[pl.ds(w * _SW + 16, _L)])

    pcv, ncv = lax.fori_loop(0, _NW, redw, (zero, zero))
    heavy = jnp.sum(ncv) > 3.0 * jnp.sum(pcv)

    @pl.when(heavy)
    def _():
        def zinit(b, _):
            hcnt[pl.ds(b * _L, _L)] = zero
            hsum[pl.ds(b * _L, _L)] = zero
            return 0

        lax.fori_loop(0, _NB, zinit, 0)

        def start(c, slot):
            sl = pl.ds(wid * _RW + c * _RC, _RC)
            pltpu.async_copy(pred_hbm.at[sl, :], pbuf.at[slot], sems[slot])
            pltpu.async_copy(gt_hbm.at[sl, :], gbuf.at[slot], sems[slot])
            pltpu.async_copy(mask_hbm.at[sl, :], mbuf.at[slot], sems[slot])

        def drain(slot):
            sl = pl.ds(0, _RC)
            pltpu.make_async_copy(
                pred_hbm.at[sl, :], pbuf.at[slot], sems[slot]).wait()
            pltpu.make_async_copy(
                gt_hbm.at[sl, :], gbuf.at[slot], sems[slot]).wait()
            pltpu.make_async_copy(
                mask_hbm.at[sl, :], mbuf.at[slot], sems[slot]).wait()

        def compute(slot, carry):
            def elem_body(j, acc):
                pc, nc, ps, ns = acc
                for r in range(_RC):
                    for u in range(2):
                        off = j * (2 * _L) + u * _L
                        p = pbuf[slot, r, pl.ds(off, _L)]
                        g = gbuf[slot, r, pl.ds(off, _L)]
                        m = mbuf[slot, r, pl.ds(off, _L)]
                        pos = g * m
                        neg = m - pos
                        y = jnp.where(g > 0.5, p, 1.0 - p)
                        loss = -_vlog(y + 1e-37) * jnp.exp(-y)
                        nl = loss * neg
                        bin_i = jnp.minimum(
                            (p * float(_NB)).astype(jnp.int32), _NB - 1)
                        idx = (_NB - 1 - bin_i) * _L + lane
                        plsc.addupdate_scatter(hcnt, [idx], neg)
                        plsc.addupdate_scatter(hsum, [idx], nl)
                        pc, nc, ps, ns = (pc + pos, nc + neg,
                                          ps + loss * pos, ns + nl)
                return (pc, nc, ps, ns)

            return lax.fori_loop(0, _COLS // (2 * _L), elem_body, carry)

        start(0, 0)

        def chunk_pair(c2, carry):
            c = 2 * c2
            start(c + 1, 1)
            drain(0)
            carry = compute(0, carry)

            @pl.when(c2 < _NCH // 2 - 1)
            def _():
                start(c + 2, 0)

            drain(1)
            return compute(1, carry)

        pc, nc, ps, ns = lax.fori_loop(
            0, _NCH // 2, chunk_pair, (zero, zero, zero, zero))

        obuf[pl.ds(0, _L)] = pc
        obuf[pl.ds(16, _L)] = nc
        obuf[pl.ds(32, _L)] = ps
        obuf[pl.ds(48, _L)] = ns

        stride = lane * _L
        for grp in range(_NB // _L):
            cacc = zero
            sacc = zero
            for l in range(_L):
                gi = stride + (grp * _L * _L + l)
                cacc = cacc + plsc.load_gather(hcnt, [gi])
                sacc = sacc + plsc.load_gather(hsum, [gi])
            obuf[pl.ds(64 + grp * _L, _L)] = cacc
            obuf[pl.ds(64 + _NB + grp * _L, _L)] = sacc

        pltpu.sync_copy(obuf, out_hbm.at[pl.ds(wid * _PW, _PW)])


@functools.partial(
    pl.kernel,
    out_type=jax.ShapeDtypeStruct((_L,), jnp.float32),
    mesh=_mesh,
    compiler_params=pltpu.CompilerParams(needs_layout_passes=False),
    scratch_types=[
        pltpu.VMEM((_NW * _SW,), jnp.float32),
        pltpu.VMEM((_NW * _PW,), jnp.float32),
        pltpu.VMEM((_L,), jnp.float32),
    ],
)
def _final(part1_hbm, part2_hbm, out_hbm, p1buf, p2buf, obuf):
    wid = lax.axis_index("s") * _NC + lax.axis_index("c")

    @pl.when(wid == 0)
    def _():
        zero = jnp.zeros((_L,), jnp.float32)
        pltpu.sync_copy(part1_hbm, p1buf)

        def redw(w, acc):
            base = w * _SW
            return tuple(
                acc[i] + p1buf[pl.ds(base + i * _L, _L)] for i in range(4))

        a1 = lax.fori_loop(0, _NW, redw, (zero, zero, zero, zero))
        pc = jnp.sum(a1[0])
        nc = jnp.sum(a1[1])
        ps = jnp.sum(a1[2])
        ns = jnp.sum(a1[3])
        heavy = nc > 3.0 * pc

        @pl.when(jnp.logical_not(heavy))
        def _():
            num = jnp.full((_L,), ps + ns, jnp.float32)
            den = jnp.full((_L,), pc + nc + 1e-6, jnp.float32)
            obuf[...] = num / den

        @pl.when(heavy)
        def _():
            pltpu.sync_copy(part2_hbm, p2buf)
            nvec = _PW // _L

            def wbody(w, accs):
                base = w * _PW
                return tuple(
                    accs[i] + p2buf[pl.ds(base + i * _L, _L)]
                    for i in range(nvec))

            accs = lax.fori_loop(0, _NW, wbody,
                                 tuple(zero for _ in range(nvec)))
            k = jnp.minimum(nc, 3.0 * pc)

            top = jnp.float32(0.0)
            cum = jnp.float32(0.0)
            for grp in range(_NB // _L):
                c16 = accs[4 + grp]
                s16 = accs[4 + _NB // _L + grp]
                excl = plsc.cumsum(c16) - c16
                take = jnp.clip(k - (cum + excl), 0.0, c16)
                top = top + jnp.sum(take * (s16 / jnp.maximum(c16, 1.0)))
                cum = cum + jnp.sum(c16)

            num = jnp.full((_L,), ps + top, jnp.float32)
            den = jnp.full((_L,), pc + k + 1e-6, jnp.float32)
            obuf[...] = num / den

        pltpu.sync_copy(obuf, out_hbm)


def kernel(pred, gt, mask):
    p = pred.reshape(_ROWS, _COLS)
    g = gt.reshape(_ROWS, _COLS)
    m = mask.reshape(_ROWS, _COLS)
    part1 = _sums(p, g, m)
    part2 = _hist(p, g, m, part1)
    out = _final(part1, part2)
    return out[0]


# speedup vs baseline: 3.7534x; 1.2780x over previous
import functools

import jax
import jax.numpy as jnp
from jax import lax
from jax.experimental import pallas as pl
from jax.experimental.pallas import tpu as pltpu
from jax.experimental.pallas import tpu_sc as plsc

_NC = 2
_NS = 16
_NW = _NC * _NS
_L = 16

_ROWS = 2048
_COLS = 2048
_RW = _ROWS // _NW
_RC = 8
_NCH = _RW // _RC

_NB = 64
_PW = 64 + 2 * _NB
_SW = 64

_LN2 = 0.6931471805599453
_LOG2E = 1.4426950408889634

_C5 = 0.04342836333161877
_C4 = -0.4048623094159427
_C3 = 1.5938845482687833
_C2 = -3.4924660425569987
_C1 = 5.046852935529714
_C0 = -2.7868055642994554 - 127.0


def _vloss(y):
    b = lax.bitcast_convert_type(y + 1e-37, jnp.int32)
    ef = (b >> 23).astype(jnp.float32)
    m = lax.bitcast_convert_type((b & 0x7FFFFF) | (127 << 23), jnp.float32)
    q = ((((_C5 * m + _C4) * m + _C3) * m + _C2) * m + _C1) * m + _C0
    return (ef + q) * (-_LN2) * jnp.exp(-y)


_mesh = plsc.VectorSubcoreMesh(core_axis_name="c", subcore_axis_name="s")


@functools.partial(
    pl.kernel,
    out_type=jax.ShapeDtypeStruct((_NW * _SW,), jnp.float32),
    mesh=_mesh,
    compiler_params=pltpu.CompilerParams(needs_layout_passes=False),
    scratch_types=[
        pltpu.VMEM((2, _RC, _COLS), jnp.float32),
        pltpu.VMEM((2, _RC, _COLS), jnp.float32),
        pltpu.VMEM((2, _RC, _COLS), jnp.float32),
        pltpu.VMEM((_SW,), jnp.float32),
        pltpu.SemaphoreType.DMA,
        pltpu.SemaphoreType.DMA,
    ],
)
def _sums(pred_hbm, gt_hbm, mask_hbm, out_hbm, pbuf, gbuf, mbuf, obuf,
          sem0, sem1):
    wid = lax.axis_index("s") * _NC + lax.axis_index("c")
    zero = jnp.zeros((_L,), jnp.float32)
    sems = (sem0, sem1)

    def start(c, slot):
        sl = pl.ds(wid * _RW + c * _RC, _RC)
        pltpu.async_copy(pred_hbm.at[sl, :], pbuf.at[slot], sems[slot])
        pltpu.async_copy(gt_hbm.at[sl, :], gbuf.at[slot], sems[slot])
        pltpu.async_copy(mask_hbm.at[sl, :], mbuf.at[slot], sems[slot])

    def drain(slot):
        sl = pl.ds(0, _RC)
        pltpu.make_async_copy(
            pred_hbm.at[sl, :], pbuf.at[slot], sems[slot]).wait()
        pltpu.make_async_copy(
            gt_hbm.at[sl, :], gbuf.at[slot], sems[slot]).wait()
        pltpu.make_async_copy(
            mask_hbm.at[sl, :], mbuf.at[slot], sems[slot]).wait()

    def compute(slot, carry):
        def elem_body(j, acc):
            pc, mc, ps, ls = acc
            for r in range(_RC):
                for u in range(2):
                    off = j * (2 * _L) + u * _L
                    p = pbuf[slot, r, pl.ds(off, _L)]
                    g = gbuf[slot, r, pl.ds(off, _L)]
                    m = mbuf[slot, r, pl.ds(off, _L)]
                    pos = g * m
                    y = jnp.where(g > 0.5, p, 1.0 - p)
                    loss = _vloss(y)
                    pc, mc, ps, ls = (pc + pos, mc + m, ps + loss * pos,
                                      ls + loss * m)
            return (pc, mc, ps, ls)

        return lax.fori_loop(0, _COLS // (2 * _L), elem_body, carry)

    start(0, 0)

    def chunk_pair(c2, carry):
        c = 2 * c2
        start(c + 1, 1)
        drain(0)
        carry = compute(0, carry)

        @pl.when(c2 < _NCH // 2 - 1)
        def _():
            start(c + 2, 0)

        drain(1)
        return compute(1, carry)

    pc, mc, ps, ls = lax.fori_loop(
        0, _NCH // 2, chunk_pair, (zero, zero, zero, zero))

    obuf[pl.ds(0, _L)] = pc
    obuf[pl.ds(16, _L)] = mc - pc
    obuf[pl.ds(32, _L)] = ps
    obuf[pl.ds(48, _L)] = ls - ps
    pltpu.sync_copy(obuf, out_hbm.at[pl.ds(wid * _SW, _SW)])


@functools.partial(
    pl.kernel,
    out_type=jax.ShapeDtypeStruct((_NW * _PW,), jnp.float32),
    mesh=_mesh,
    compiler_params=pltpu.CompilerParams(needs_layout_passes=False),
    scratch_types=[
        pltpu.VMEM((2, _RC, _COLS), jnp.float32),
        pltpu.VMEM((2, _RC, _COLS), jnp.float32),
        pltpu.VMEM((2, _RC, _COLS), jnp.float32),
        pltpu.VMEM((_NB * _L,), jnp.float32),
        pltpu.VMEM((_NB * _L,), jnp.float32),
        pltpu.VMEM((_PW,), jnp.float32),
        pltpu.VMEM((_NW * _SW,), jnp.float32),
        pltpu.SemaphoreType.DMA,
        pltpu.SemaphoreType.DMA,
    ],
)
def _hist(pred_hbm, gt_hbm, mask_hbm, part1_hbm, out_hbm, pbuf, gbuf, mbuf,
          hcnt, hsum, obuf, p1buf, sem0, sem1):
    wid = lax.axis_index("s") * _NC + lax.axis_index("c")
    lane = lax.iota(jnp.int32, _L)
    zero = jnp.zeros((_L,), jnp.float32)
    sems = (sem0, sem1)

    pltpu.sync_copy(part1_hbm, p1buf)

    def redw(w, acc):
        pcv, ncv = acc
        return (pcv + p1buf[pl.ds(w * _SW, _L)],
                ncv + p1buf[pl.ds(w * _SW + 16, _L)])

    pcv, ncv = lax.fori_loop(0, _NW, redw, (zero, zero))
    heavy = jnp.sum(ncv) > 3.0 * jnp.sum(pcv)

    @pl.when(heavy)
    def _():
        def zinit(b, _):
            hcnt[pl.ds(b * _L, _L)] = zero
            hsum[pl.ds(b * _L, _L)] = zero
            return 0

        lax.fori_loop(0, _NB, zinit, 0)

        def start(c, slot):
            sl = pl.ds(wid * _RW + c * _RC, _RC)
            pltpu.async_copy(pred_hbm.at[sl, :], pbuf.at[slot], sems[slot])
            pltpu.async_copy(gt_hbm.at[sl, :], gbuf.at[slot], sems[slot])
            pltpu.async_copy(mask_hbm.at[sl, :], mbuf.at[slot], sems[slot])

        def drain(slot):
            sl = pl.ds(0, _RC)
            pltpu.make_async_copy(
                pred_hbm.at[sl, :], pbuf.at[slot], sems[slot]).wait()
            pltpu.make_async_copy(
                gt_hbm.at[sl, :], gbuf.at[slot], sems[slot]).wait()
            pltpu.make_async_copy(
                mask_hbm.at[sl, :], mbuf.at[slot], sems[slot]).wait()

        def compute(slot, carry):
            def elem_body(j, acc):
                pc, nc, ps, ns = acc
                for r in range(_RC):
                    for u in range(2):
                        off = j * (2 * _L) + u * _L
                        p = pbuf[slot, r, pl.ds(off, _L)]
                        g = gbuf[slot, r, pl.ds(off, _L)]
                        m = mbuf[slot, r, pl.ds(off, _L)]
                        pos = g * m
                        neg = m - pos
                        y = jnp.where(g > 0.5, p, 1.0 - p)
                        loss = _vloss(y)
                        nl = loss * neg
                        bin_i = jnp.minimum(
                            (p * float(_NB)).astype(jnp.int32), _NB - 1)
                        idx = (_NB - 1 - bin_i) * _L + lane
                        plsc.addupdate_scatter(hcnt, [idx], neg)
                        plsc.addupdate_scatter(hsum, [idx], nl)
                        pc, nc, ps, ns = (pc + pos, nc + neg,
                                          ps + loss * pos, ns + nl)
                return (pc, nc, ps, ns)

            return lax.fori_loop(0, _COLS // (2 * _L), elem_body, carry)

        start(0, 0)

        def chunk_pair(c2, carry):
            c = 2 * c2
            start(c + 1, 1)
            drain(0)
            carry = compute(0, carry)

            @pl.when(c2 < _NCH // 2 - 1)
            def _():
                start(c + 2, 0)

            drain(1)
            return compute(1, carry)

        pc, nc, ps, ns = lax.fori_loop(
            0, _NCH // 2, chunk_pair, (zero, zero, zero, zero))

        obuf[pl.ds(0, _L)] = pc
        obuf[pl.ds(16, _L)] = nc
        obuf[pl.ds(32, _L)] = ps
        obuf[pl.ds(48, _L)] = ns

        stride = lane * _L
        for grp in range(_NB // _L):
            cacc = zero
            sacc = zero
            for l in range(_L):
                gi = stride + (grp * _L * _L + l)
                cacc = cacc + plsc.load_gather(hcnt, [gi])
                sacc = sacc + plsc.load_gather(hsum, [gi])
            obuf[pl.ds(64 + grp * _L, _L)] = cacc
            obuf[pl.ds(64 + _NB + grp * _L, _L)] = sacc

        pltpu.sync_copy(obuf, out_hbm.at[pl.ds(wid * _PW, _PW)])


@functools.partial(
    pl.kernel,
    out_type=jax.ShapeDtypeStruct((_L,), jnp.float32),
    mesh=_mesh,
    compiler_params=pltpu.CompilerParams(needs_layout_passes=False),
    scratch_types=[
        pltpu.VMEM((_NW * _SW,), jnp.float32),
        pltpu.VMEM((_NW * _PW,), jnp.float32),
        pltpu.VMEM((_L,), jnp.float32),
    ],
)
def _final(part1_hbm, part2_hbm, out_hbm, p1buf, p2buf, obuf):
    wid = lax.axis_index("s") * _NC + lax.axis_index("c")

    @pl.when(wid == 0)
    def _():
        zero = jnp.zeros((_L,), jnp.float32)
        pltpu.sync_copy(part1_hbm, p1buf)

        def redw(w, acc):
            base = w * _SW
            return tuple(
                acc[i] + p1buf[pl.ds(base + i * _L, _L)] for i in range(4))

        a1 = lax.fori_loop(0, _NW, redw, (zero, zero, zero, zero))
        pc = jnp.sum(a1[0])
        nc = jnp.sum(a1[1])
        ps = jnp.sum(a1[2])
        ns = jnp.sum(a1[3])
        heavy = nc > 3.0 * pc

        @pl.when(jnp.logical_not(heavy))
        def _():
            num = jnp.full((_L,), ps + ns, jnp.float32)
            den = jnp.full((_L,), pc + nc + 1e-6, jnp.float32)
            obuf[...] = num / den

        @pl.when(heavy)
        def _():
            pltpu.sync_copy(part2_hbm, p2buf)
            nvec = _PW // _L

            def wbody(w, accs):
                base = w * _PW
                return tuple(
                    accs[i] + p2buf[pl.ds(base + i * _L, _L)]
                    for i in range(nvec))

            accs = lax.fori_loop(0, _NW, wbody,
                                 tuple(zero for _ in range(nvec)))
            k = jnp.minimum(nc, 3.0 * pc)

            top = jnp.float32(0.0)
            cum = jnp.float32(0.0)
            for grp in range(_NB // _L):
                c16 = accs[4 + grp]
                s16 = accs[4 + _NB // _L + grp]
                excl = plsc.cumsum(c16) - c16
                take = jnp.clip(k - (cum + excl), 0.0, c16)
                top = top + jnp.sum(take * (s16 / jnp.maximum(c16, 1.0)))
                cum = cum + jnp.sum(c16)

            num = jnp.full((_L,), ps + top, jnp.float32)
            den = jnp.full((_L,), pc + k + 1e-6, jnp.float32)
            obuf[...] = num / den

        pltpu.sync_copy(obuf, out_hbm)


def kernel(pred, gt, mask):
    p = pred.reshape(_ROWS, _COLS)
    g = gt.reshape(_ROWS, _COLS)
    m = mask.reshape(_ROWS, _COLS)
    part1 = _sums(p, g, m)
    part2 = _hist(p, g, m, part1)
    out = _final(part1, part2)
    return out[0]
